# baseline (device time: 46298 ns/iter reference)
import jax
import jax.numpy as jnp
from jax import lax
from jax.experimental import pallas as pl
from jax.experimental.pallas import tpu as pltpu

N_DEV = 4
M_CHUNK = 512
D = 512
EPS = 1e-6


def kernel(partial, gamma):
    gamma2d = gamma.reshape(1, D)

    def body(x_ref, g_ref, out_ref, send_ref, recv_ref, send_sems, recv_sems):
        my = lax.axis_index("i")
        left = lax.rem(my + N_DEV - 1, N_DEV)
        right = lax.rem(my + 1, N_DEV)

        def chunk(c):
            return x_ref[0, pl.ds(c * M_CHUNK, M_CHUNK), :]

        barrier_sem = pltpu.get_barrier_semaphore()
        for nbr in (left, right):
            pl.semaphore_signal(
                barrier_sem, inc=1,
                device_id=(nbr,), device_id_type=pl.DeviceIdType.MESH,
            )
        pl.semaphore_wait(barrier_sem, 2)

        send_ref[0] = chunk(lax.rem(my + N_DEV - 1, N_DEV))

        for h in range(N_DEV - 1):
            rdma = pltpu.make_async_remote_copy(
                src_ref=send_ref.at[h],
                dst_ref=recv_ref.at[h],
                send_sem=send_sems.at[h],
                recv_sem=recv_sems.at[h],
                device_id=(right,),
                device_id_type=pl.DeviceIdType.MESH,
            )
            rdma.start()
            rdma.wait()
            c = lax.rem(my + 2 * N_DEV - 2 - h, N_DEV)
            acc = recv_ref[h] + chunk(c)
            if h < N_DEV - 2:
                send_ref[h + 1] = acc
            else:
                ms = jnp.mean(acc * acc, axis=-1, keepdims=True)
                out_ref[...] = acc * lax.rsqrt(ms + EPS) * g_ref[...]

    return pl.pallas_call(
        body,
        out_shape=jax.ShapeDtypeStruct((M_CHUNK, D), jnp.float32),
        in_specs=[
            pl.BlockSpec(memory_space=pltpu.VMEM),
            pl.BlockSpec(memory_space=pltpu.VMEM),
        ],
        out_specs=pl.BlockSpec(memory_space=pltpu.VMEM),
        scratch_shapes=[
            pltpu.VMEM((N_DEV - 1, M_CHUNK, D), jnp.float32),
            pltpu.VMEM((N_DEV - 1, M_CHUNK, D), jnp.float32),
            pltpu.SemaphoreType.DMA((N_DEV - 1,)),
            pltpu.SemaphoreType.DMA((N_DEV - 1,)),
        ],
        compiler_params=pltpu.CompilerParams(collective_id=0),
    )(partial, gamma2d)


# device time: 25867 ns/iter; 1.7898x vs baseline; 1.7898x over previous
import jax
import jax.numpy as jnp
from jax import lax
from jax.experimental import pallas as pl
from jax.experimental.pallas import tpu as pltpu

N_DEV = 4
M_CHUNK = 512
D = 512
DH = D // 2
EPS = 1e-6


def kernel(partial, gamma):
    gamma2d = gamma.reshape(1, D)

    def body(x_ref, g_ref, out_ref, rp1, rp2, stage,
             p1_send, p1_recv, p2_send, p2_recv):
        my = lax.axis_index("i")
        pa = my ^ 1
        pb = 3 - my

        barrier_sem = pltpu.get_barrier_semaphore()
        for nbr in (pa, pb):
            pl.semaphore_signal(
                barrier_sem, inc=1,
                device_id=(nbr,), device_id_type=pl.DeviceIdType.MESH,
            )
        pl.semaphore_wait(barrier_sem, 2)

        def src(c, col0):
            return x_ref.at[0, pl.ds(c * M_CHUNK, M_CHUNK),
                            pl.ds(col0, DH)]

        def p1_rdma(slot, c, col0, target):
            return pltpu.make_async_remote_copy(
                src_ref=src(c, col0),
                dst_ref=rp1.at[slot],
                send_sem=p1_send.at[slot],
                recv_sem=p1_recv.at[slot],
                device_id=(target,),
                device_id_type=pl.DeviceIdType.MESH,
            )

        rB = p1_rdma(0, 3 - pa, 0, pa)
        rD = p1_rdma(2, pb ^ 1, DH, pb)
        rA = p1_rdma(1, pa, 0, pa)
        rC = p1_rdma(3, pb, DH, pb)
        rB.start()
        rD.start()
        rA.start()
        rC.start()

        def chunk(c, col_lo, col_hi):
            return x_ref[0, pl.ds(c * M_CHUNK, M_CHUNK), col_lo:col_hi]

        def p2_rdma(slot, target):
            return pltpu.make_async_remote_copy(
                src_ref=stage.at[slot],
                dst_ref=rp2.at[slot],
                send_sem=p2_send.at[slot],
                recv_sem=p2_recv.at[slot],
                device_id=(target,),
                device_id_type=pl.DeviceIdType.MESH,
            )

        rB.wait_recv()
        stage[0] = chunk(3 - my, 0, DH) + rp1[0]
        r3 = p2_rdma(0, pb)
        r3.start()

        rD.wait_recv()
        stage[1] = chunk(my ^ 1, DH, D) + rp1[2]
        r4 = p2_rdma(1, pa)
        r4.start()

        rA.wait_recv()
        a1 = chunk(my, 0, DH) + rp1[1]
        rC.wait_recv()
        a2 = chunk(my, DH, D) + rp1[3]

        r3.wait_recv()
        r4.wait_recv()
        y = jnp.concatenate([a1 + rp2[0], a2 + rp2[1]], axis=1)
        ms = jnp.mean(y * y, axis=-1, keepdims=True)
        out_ref[...] = y * lax.rsqrt(ms + EPS) * g_ref[...]

        for r in (rB, rD, rA, rC, r3, r4):
            r.wait_send()

    return pl.pallas_call(
        body,
        out_shape=jax.ShapeDtypeStruct((M_CHUNK, D), jnp.float32),
        in_specs=[
            pl.BlockSpec(memory_space=pltpu.VMEM),
            pl.BlockSpec(memory_space=pltpu.VMEM),
        ],
        out_specs=pl.BlockSpec(memory_space=pltpu.VMEM),
        scratch_shapes=[
            pltpu.VMEM((4, M_CHUNK, DH), jnp.float32),
            pltpu.VMEM((2, M_CHUNK, DH), jnp.float32),
            pltpu.VMEM((2, M_CHUNK, DH), jnp.float32),
            pltpu.SemaphoreType.DMA((4,)),
            pltpu.SemaphoreType.DMA((4,)),
            pltpu.SemaphoreType.DMA((2,)),
            pltpu.SemaphoreType.DMA((2,)),
        ],
        compiler_params=pltpu.CompilerParams(collective_id=0),
    )(partial, gamma2d)
